# Initial kernel scaffold; baseline (speedup 1.0000x reference)
import jax
import jax.numpy as jnp
from jax.experimental import pallas as pl


def kernel(x, edge_index, edge_attr, u, batch, W_edge, b_edge, gamma_e, beta_e, W_n1, b_n1, W_n2, b_n2, gamma_n, beta_n, W_g, b_g, gamma_g, beta_g):
    raise NotImplementedError("write your pallas kernel here")


# speedup vs baseline: 2.4435x; 2.4435x over previous
import functools

import jax
import jax.numpy as jnp
from jax import lax
from jax.experimental import pallas as pl
from jax.experimental.pallas import tpu as pltpu
from jax.experimental.pallas import tpu_sc as plsc

_NC, _NS = 2, 16
_NW = _NC * _NS
_CH = 128
_TE = 4000
_TN = 2000
_HALF = 26000
_ACC = 26624
_TILE_ROWS = _ACC // _NS
_ZCH = _TILE_ROWS // 2
_TRASH = 26016

_HBLK = _HALF // _TN

_INV_SQRT2 = 0.7071067811865476


def _gelu(h):
    return 0.5 * h * (1.0 + lax.erf(h * _INV_SQRT2))


def _ln(v, gamma, beta):
    mean = jnp.mean(v, axis=-1, keepdims=True)
    var = jnp.mean((v - mean) ** 2, axis=-1, keepdims=True)
    return (v - mean) / jnp.sqrt(var + 1e-5) * gamma + beta


def _dot(a, b):
    return lax.dot_general(a, b, (((1,), (0,)), ((), ())),
                           preferred_element_type=jnp.float32,
                           precision=lax.Precision.HIGHEST)


def _dot_t(a, b):
    return lax.dot_general(a, b, (((0,), (0,)), ((), ())),
                           preferred_element_type=jnp.float32,
                           precision=lax.Precision.HIGHEST)


def _starts_body(batch_ref, out_ref):
    i = pl.program_id(0)

    @pl.when(i == 0)
    def _():
        out_ref[...] = jnp.zeros_like(out_ref)

    b = batch_ref[...]
    lt = (b < lax.broadcasted_iota(jnp.int32, (1, 16), 1)).astype(jnp.int32)
    out_ref[...] += jnp.sum(lt, axis=0, keepdims=True)


def _starts_call(batch2d):
    n = batch2d.shape[0]
    return pl.pallas_call(
        _starts_body,
        grid=(n // _TN,),
        in_specs=[pl.BlockSpec((_TN, 1), lambda i: (i, 0))],
        out_specs=pl.BlockSpec((1, 16), lambda i: (0, 0)),
        out_shape=jax.ShapeDtypeStruct((1, 16), jnp.int32),
    )(batch2d)


def _sc_gather(x, row2d, col2d):
    n_ch = row2d.shape[0]
    e = n_ch * _CH
    q, r = divmod(n_ch, _NW)
    mesh = plsc.VectorSubcoreMesh(core_axis_name="c", subcore_axis_name="s")

    @functools.partial(
        pl.kernel, mesh=mesh,
        out_type=(jax.ShapeDtypeStruct((e, 32), jnp.float32),
                  jax.ShapeDtypeStruct((e, 32), jnp.float32)),
        scratch_types=[
            pltpu.VMEM((1, _CH), jnp.int32),
            pltpu.VMEM((1, _CH), jnp.int32),
            pltpu.VMEM((_CH, 32), jnp.float32),
            pltpu.VMEM((_CH, 32), jnp.float32),
            pltpu.SemaphoreType.DMA,
            pltpu.SemaphoreType.DMA,
        ],
        compiler_params=pltpu.CompilerParams(use_tc_tiling_on_sc=False),
    )
    def gk(x_hbm, row_hbm, col_hbm, xr_out, xc_out,
           idx_r, idx_c, buf_r, buf_c, sem_r, sem_c):
        c = lax.axis_index("c")
        s = lax.axis_index("s")
        w = s * _NC + c
        nch = q + jnp.where(w < r, 1, 0)
        base = w * q + jnp.minimum(w, r)

        def body(i, carry):
            ch = base + i
            pltpu.sync_copy(row_hbm.at[pl.ds(ch, 1)], idx_r)
            pltpu.sync_copy(col_hbm.at[pl.ds(ch, 1)], idx_c)
            cp1 = pltpu.async_copy(x_hbm.at[idx_r.at[0]], buf_r, sem_r)
            cp2 = pltpu.async_copy(x_hbm.at[idx_c.at[0]], buf_c, sem_c)
            cp1.wait()
            cp2.wait()
            pltpu.sync_copy(buf_r, xr_out.at[pl.ds(ch * _CH, _CH)])
            pltpu.sync_copy(buf_c, xc_out.at[pl.ds(ch * _CH, _CH)])
            return carry

        lax.fori_loop(0, nch, body, 0)

    return gk(x, row2d, col2d)


def _edge_body(xr_ref, xc_ref, ea_ref, row_ref, starts_ref, u_ref,
               we_ref, be_ref, ge_ref, bte_ref, wn1_ref, bn1_ref,
               eout_ref, nh_ref):
    xr = xr_ref[...]
    xc = xc_ref[...]
    ea = ea_ref[...]
    row = row_ref[...]
    starts = starts_ref[...]
    ge = (row >= starts).astype(jnp.int32)
    brow = jnp.sum(ge, axis=1, keepdims=True) - 1
    oh = (brow == lax.broadcasted_iota(jnp.int32, (1, 16), 1)).astype(jnp.float32)
    usel = _dot(oh, u_ref[...])
    e_in = jnp.concatenate([xr, xc, ea, usel], axis=1)
    e_h = _gelu(_dot(e_in, we_ref[...]) + be_ref[...])
    e_out = _ln(e_h + ea, ge_ref[...], bte_ref[...])
    n_in = jnp.concatenate([xc, e_out], axis=1)
    n_h = _gelu(_dot(n_in, wn1_ref[...]) + bn1_ref[...])
    eout_ref[...] = e_out
    nh_ref[...] = n_h


def _edge_call(xr, xc, ea, row_col, starts, u, W_edge, b_edge,
               gamma_e, beta_e, W_n1, b_n1):
    e = xr.shape[0]
    blk = lambda i: (i, 0)
    full = lambda i: (0, 0)
    return pl.pallas_call(
        _edge_body,
        grid=(e // _TE,),
        in_specs=[
            pl.BlockSpec((_TE, 32), blk),
            pl.BlockSpec((_TE, 32), blk),
            pl.BlockSpec((_TE, 32), blk),
            pl.BlockSpec((_TE, 1), blk),
            pl.BlockSpec((1, 16), full),
            pl.BlockSpec((16, 32), full),
            pl.BlockSpec((128, 32), full),
            pl.BlockSpec((1, 32), full),
            pl.BlockSpec((1, 32), full),
            pl.BlockSpec((1, 32), full),
            pl.BlockSpec((64, 32), full),
            pl.BlockSpec((1, 32), full),
        ],
        out_specs=[pl.BlockSpec((_TE, 32), blk), pl.BlockSpec((_TE, 32), blk)],
        out_shape=[jax.ShapeDtypeStruct((e, 32), jnp.float32),
                   jax.ShapeDtypeStruct((e, 32), jnp.float32)],
    )(xr, xc, ea, row_col, starts, u, W_edge, b_edge, gamma_e, beta_e, W_n1, b_n1)


def _sc_scatter(nh, row2d):
    n_ch = row2d.shape[0]
    q, r = divmod(n_ch, _NS)
    mesh = plsc.VectorSubcoreMesh(core_axis_name="c", subcore_axis_name="s")
    z2d = jnp.zeros((_ZCH, 32), jnp.float32)
    z1d = jnp.zeros((_ZCH, 1), jnp.float32)
    ones = jnp.ones((_CH, 1), jnp.float32)

    @functools.partial(
        pl.kernel, mesh=mesh,
        out_type=(jax.ShapeDtypeStruct((_NC, _ACC, 32), jnp.float32),
                  jax.ShapeDtypeStruct((_NC, _ACC, 1), jnp.float32)),
        scratch_types=[
            pltpu.VMEM((1, _CH), jnp.int32),
            pltpu.VMEM((1, _CH), jnp.int32),
            pltpu.VMEM((_CH, 32), jnp.float32),
            pltpu.VMEM((_CH, 1), jnp.float32),
            pltpu.VMEM((_ZCH, 32), jnp.float32),
            pltpu.VMEM((_ZCH, 1), jnp.float32),
            pltpu.VMEM_SHARED((_ACC, 32), jnp.float32),
            pltpu.VMEM_SHARED((_ACC, 1), jnp.float32),
        ],
        compiler_params=pltpu.CompilerParams(use_tc_tiling_on_sc=False),
    )
    def sk(nh_hbm, row_hbm, z2d_hbm, z1d_hbm, ones_hbm, psum_out, pcnt_out,
           idx_v, idxt_v, dat_v, ones_v, buf2d, buf1d, acc_sh, cnt_sh):
        c = lax.axis_index("c")
        s = lax.axis_index("s")
        tb = s * _TILE_ROWS
        lo = c * _HALF
        pltpu.sync_copy(z2d_hbm, buf2d)
        pltpu.sync_copy(z1d_hbm, buf1d)
        pltpu.sync_copy(ones_hbm, ones_v)
        for k in range(_TILE_ROWS // _ZCH):
            pltpu.sync_copy(buf2d, acc_sh.at[pl.ds(tb + k * _ZCH, _ZCH)])
            pltpu.sync_copy(buf1d, cnt_sh.at[pl.ds(tb + k * _ZCH, _ZCH)])
        plsc.subcore_barrier()

        nch = q + jnp.where(s < r, 1, 0)
        base = s * q + jnp.minimum(s, r)

        def body(i, carry):
            ch = base + i
            pltpu.sync_copy(row_hbm.at[pl.ds(ch, 1)], idx_v)
            pltpu.sync_copy(nh_hbm.at[pl.ds(ch * _CH, _CH)], dat_v)
            for k in range(_CH // 16):
                t = idx_v[0, pl.ds(k * 16, 16)] - lo
                valid = (t >= 0) & (t < _HALF)
                idxt_v[0, pl.ds(k * 16, 16)] = jnp.where(valid, t, _TRASH)
            pltpu.sync_copy(dat_v, acc_sh.at[idxt_v.at[0]], add=True)
            pltpu.sync_copy(ones_v, cnt_sh.at[idxt_v.at[0]], add=True)
            return carry

        lax.fori_loop(0, nch, body, 0)
        plsc.subcore_barrier()

        for k in range(_TILE_ROWS // _ZCH):
            pltpu.sync_copy(acc_sh.at[pl.ds(tb + k * _ZCH, _ZCH)], buf2d)
            pltpu.sync_copy(buf2d, psum_out.at[c, pl.ds(tb + k * _ZCH, _ZCH)])
            pltpu.sync_copy(cnt_sh.at[pl.ds(tb + k * _ZCH, _ZCH)], buf1d)
            pltpu.sync_copy(buf1d, pcnt_out.at[c, pl.ds(tb + k * _ZCH, _ZCH)])

    return sk(nh, row2d, z2d, z1d, ones)


def _node_body(s_ref, c_ref, x_ref, batch_ref, u_ref,
               wn2_ref, bn2_ref, gn_ref, btn_ref, wg_ref, bg_ref, gg_ref, btg_ref,
               xout_ref, uout_ref, gx_acc, cnt_acc):
    i = pl.program_id(0)
    nsteps = pl.num_programs(0)

    @pl.when(i == 0)
    def _():
        gx_acc[...] = jnp.zeros_like(gx_acc)
        cnt_acc[...] = jnp.zeros_like(cnt_acc)

    ssum = s_ref[0]
    cnt = c_ref[0]
    agg = ssum / jnp.maximum(cnt, 1.0)
    bcol = batch_ref[...]
    oh = (bcol == lax.broadcasted_iota(jnp.int32, (1, 16), 1)).astype(jnp.float32)
    ub = _dot(oh, u_ref[...])
    n2 = _gelu(_dot(jnp.concatenate([agg, ub], axis=1), wn2_ref[...]) + bn2_ref[...])
    xo = _ln(n2 + x_ref[...], gn_ref[...], btn_ref[...])
    xout_ref[...] = xo
    gx_acc[...] += _dot_t(oh, xo)
    cnt_acc[...] += _dot_t(oh, jnp.ones((oh.shape[0], 1), jnp.float32))

    @pl.when(i == nsteps - 1)
    def _():
        uu = u_ref[...]
        gxm = gx_acc[...] / jnp.maximum(cnt_acc[...], 1.0)
        g_h = _gelu(_dot(jnp.concatenate([uu, gxm], axis=1), wg_ref[...]) + bg_ref[...])
        uout_ref[...] = _ln(g_h + uu, gg_ref[...], btg_ref[...])


def _node_call(psum, pcnt, x, batch2d, u, W_n2, b_n2, gamma_n, beta_n,
               W_g, b_g, gamma_g, beta_g):
    n = x.shape[0]
    split = lambda i: (i // _HBLK, i - _HBLK * (i // _HBLK), 0)
    blk = lambda i: (i, 0)
    full = lambda i: (0, 0)
    return pl.pallas_call(
        _node_body,
        grid=(n // _TN,),
        in_specs=[
            pl.BlockSpec((1, _TN, 32), split),
            pl.BlockSpec((1, _TN, 1), split),
            pl.BlockSpec((_TN, 32), blk),
            pl.BlockSpec((_TN, 1), blk),
            pl.BlockSpec((16, 32), full),
            pl.BlockSpec((64, 32), full),
            pl.BlockSpec((1, 32), full),
            pl.BlockSpec((1, 32), full),
            pl.BlockSpec((1, 32), full),
            pl.BlockSpec((64, 32), full),
            pl.BlockSpec((1, 32), full),
            pl.BlockSpec((1, 32), full),
            pl.BlockSpec((1, 32), full),
        ],
        out_specs=[pl.BlockSpec((_TN, 32), blk), pl.BlockSpec((16, 32), full)],
        out_shape=[jax.ShapeDtypeStruct((n, 32), jnp.float32),
                   jax.ShapeDtypeStruct((16, 32), jnp.float32)],
        scratch_shapes=[pltpu.VMEM((16, 32), jnp.float32),
                        pltpu.VMEM((16, 1), jnp.float32)],
    )(psum, pcnt, x, batch2d, u, W_n2, b_n2, gamma_n, beta_n,
      W_g, b_g, gamma_g, beta_g)


def kernel(x, edge_index, edge_attr, u, batch,
           W_edge, b_edge, gamma_e, beta_e,
           W_n1, b_n1, W_n2, b_n2, gamma_n, beta_n,
           W_g, b_g, gamma_g, beta_g):
    n, h = x.shape
    e = edge_attr.shape[0]
    row = edge_index[0]
    col = edge_index[1]
    row2d = row.reshape(e // _CH, _CH)
    col2d = col.reshape(e // _CH, _CH)

    starts = _starts_call(batch.reshape(n, 1))
    xr, xc = _sc_gather(x, row2d, col2d)
    e_out, n_h = _edge_call(
        xr, xc, edge_attr, row.reshape(e, 1), starts, u,
        W_edge, b_edge.reshape(1, h), gamma_e.reshape(1, h),
        beta_e.reshape(1, h), W_n1, b_n1.reshape(1, h))
    psum, pcnt = _sc_scatter(n_h, row2d)
    x_out, u_out = _node_call(
        psum, pcnt, x, batch.reshape(n, 1), u,
        W_n2, b_n2.reshape(1, h), gamma_n.reshape(1, h), beta_n.reshape(1, h),
        W_g, b_g.reshape(1, h), gamma_g.reshape(1, h), beta_g.reshape(1, h))
    return x_out, e_out, u_out
